# Initial kernel scaffold; baseline (speedup 1.0000x reference)
import jax
import jax.numpy as jnp
from jax.experimental import pallas as pl


def kernel(inputs, embedding_table, dense_W, dense_b):
    raise NotImplementedError("write your pallas kernel here")


# speedup vs baseline: 1.1042x; 1.1042x over previous
import functools

import jax
import jax.numpy as jnp
from jax import lax
from jax.experimental import pallas as pl
from jax.experimental.pallas import tpu as pltpu
from jax.experimental.pallas import tpu_sc as plsc

V = 253854
EMB = 300
CTX = 10
B = 1024

TV = 2048
NV = (V + TV - 1) // TV

NC, NS = 2, 16
NW = NC * NS
NROWS = B * CTX
RPW = NROWS // NW
CHUNK = 80
NCH = RPW // CHUNK


GEMB = 384


def _sc_gather(table, tail_pad, idx_flat):
    mesh = plsc.VectorSubcoreMesh(core_axis_name="c", subcore_axis_name="s")

    @functools.partial(
        pl.kernel,
        mesh=mesh,
        out_type=jax.ShapeDtypeStruct((NROWS, GEMB), jnp.float32),
        scratch_types=[
            pltpu.VMEM((NCH, CHUNK), jnp.int32),
            pltpu.VMEM((RPW, GEMB), jnp.float32),
            pltpu.SemaphoreType.DMA,
        ],
    )
    def k(table_hbm, tail_hbm, idx_hbm, out_hbm, idx_v, rows_v, sem):
        wid = lax.axis_index("s") * NC + lax.axis_index("c")
        base = wid * RPW
        for c in range(NCH):
            pltpu.sync_copy(idx_hbm.at[pl.ds(base + c * CHUNK, CHUNK)],
                            idx_v.at[c])
        cps = []
        for c in range(NCH):
            rsel = pl.ds(c * CHUNK, CHUNK)
            for h in range(2):
                cps.append(pltpu.async_copy(
                    table_hbm.at[idx_v.at[c], pl.ds(h * 128, 128)],
                    rows_v.at[rsel, pl.ds(h * 128, 128)], sem))
            cps.append(pltpu.async_copy(
                tail_hbm.at[idx_v.at[c]],
                rows_v.at[rsel, pl.ds(256, 128)], sem))
        for cp in cps:
            cp.wait()
        pltpu.sync_copy(rows_v, out_hbm.at[pl.ds(base, RPW)])

    return k(table, tail_pad, idx_flat)


def _p1(g3, w, b2):
    def body(g_ref, w_ref, b_ref, m_out, d_out, a_out, m_sc, d_sc, a_sc):
        j = pl.program_id(0)

        @pl.when(j == 0)
        def _():
            a_sc[...] = jnp.sum(g_ref[...], axis=0)[:, :EMB]
            m_sc[...] = jnp.full((B, 1), -jnp.inf, jnp.float32)
            d_sc[...] = jnp.zeros((B, 1), jnp.float32)

        logits = lax.dot_general(
            a_sc[...], w_ref[...], (((1,), (0,)), ((), ())),
            preferred_element_type=jnp.float32,
        ) * (1.0 / CTX) + b_ref[...]
        col = j * TV + lax.broadcasted_iota(jnp.int32, (1, TV), 1)
        logits = jnp.where(col < V, logits, -jnp.inf)
        mj = jnp.max(logits, axis=1, keepdims=True)
        nm = jnp.maximum(m_sc[...], mj)
        d_sc[...] = (d_sc[...] * jnp.exp(m_sc[...] - nm)
                     + jnp.sum(jnp.exp(logits - nm), axis=1, keepdims=True))
        m_sc[...] = nm

        @pl.when(j == NV - 1)
        def _():
            m_out[...] = m_sc[...]
            d_out[...] = d_sc[...]
            a_out[...] = a_sc[...]

    return pl.pallas_call(
        body,
        grid=(NV,),
        in_specs=[
            pl.BlockSpec((CTX, B, GEMB), lambda j: (0, 0, 0)),
            pl.BlockSpec((EMB, TV), lambda j: (0, j)),
            pl.BlockSpec((1, TV), lambda j: (0, j)),
        ],
        out_specs=[
            pl.BlockSpec((B, 1), lambda j: (0, 0)),
            pl.BlockSpec((B, 1), lambda j: (0, 0)),
            pl.BlockSpec((B, EMB), lambda j: (0, 0)),
        ],
        out_shape=[
            jax.ShapeDtypeStruct((B, 1), jnp.float32),
            jax.ShapeDtypeStruct((B, 1), jnp.float32),
            jax.ShapeDtypeStruct((B, EMB), jnp.float32),
        ],
        scratch_shapes=[
            pltpu.VMEM((B, 1), jnp.float32),
            pltpu.VMEM((B, 1), jnp.float32),
            pltpu.VMEM((B, EMB), jnp.float32),
        ],
        compiler_params=pltpu.CompilerParams(
            dimension_semantics=("arbitrary",)),
    )(g3, w, b2)


def _p2(a_sum, w, b2, m, d):
    def body(a_ref, w_ref, b_ref, m_ref, d_ref, o_ref):
        logits = lax.dot_general(
            a_ref[...], w_ref[...], (((1,), (0,)), ((), ())),
            preferred_element_type=jnp.float32,
        ) * (1.0 / CTX) + b_ref[...]
        o_ref[...] = jnp.exp(logits - m_ref[...]) * (1.0 / d_ref[...])

    return pl.pallas_call(
        body,
        grid=(NV,),
        in_specs=[
            pl.BlockSpec((B, EMB), lambda j: (0, 0)),
            pl.BlockSpec((EMB, TV), lambda j: (0, j)),
            pl.BlockSpec((1, TV), lambda j: (0, j)),
            pl.BlockSpec((B, 1), lambda j: (0, 0)),
            pl.BlockSpec((B, 1), lambda j: (0, 0)),
        ],
        out_specs=pl.BlockSpec((B, TV), lambda j: (0, j)),
        out_shape=jax.ShapeDtypeStruct((B, V), jnp.float32),
        compiler_params=pltpu.CompilerParams(
            dimension_semantics=("arbitrary",)),
    )(a_sum, w, b2, m, d)


def kernel(inputs, embedding_table, dense_W, dense_b):
    idx_flat = inputs.astype(jnp.int32).T.reshape(-1)
    tail_pad = jnp.pad(embedding_table[:, 256:], ((0, 0), (0, 128 - (EMB - 256))))
    g = _sc_gather(embedding_table, tail_pad, idx_flat)
    g3 = g.reshape(CTX, B, GEMB)
    b2 = dense_b.reshape(1, V)
    m, d, a_sum = _p1(g3, dense_W, b2)
    return _p2(a_sum, dense_W, b2, m, d)


# speedup vs baseline: 1.6442x; 1.4891x over previous
import functools

import jax
import jax.numpy as jnp
from jax import lax
from jax.experimental.layout import Format, Layout, with_layout_constraint
from jax.experimental import pallas as pl
from jax.experimental.pallas import tpu as pltpu
from jax.experimental.pallas import tpu_sc as plsc

V = 253854
EMB = 300
CTX = 10
B = 1024

TV = 2048
NV = (V + TV - 1) // TV

NC, NS = 2, 16
NW = NC * NS
NROWS = B * CTX
RPW = NROWS // NW
CHUNK = 80
NCH = RPW // CHUNK


GEMB = 384


def _sc_gather(table, tail_pad, idx_flat):
    mesh = plsc.VectorSubcoreMesh(core_axis_name="c", subcore_axis_name="s")

    @functools.partial(
        pl.kernel,
        mesh=mesh,
        out_type=jax.ShapeDtypeStruct((NROWS, GEMB), jnp.float32),
        scratch_types=[
            pltpu.VMEM((NCH, CHUNK), jnp.int32),
            pltpu.VMEM((RPW, GEMB), jnp.float32),
            pltpu.SemaphoreType.DMA,
        ],
    )
    def k(table_hbm, tail_hbm, idx_hbm, out_hbm, idx_v, rows_v, sem):
        wid = lax.axis_index("s") * NC + lax.axis_index("c")
        base = wid * RPW
        for c in range(NCH):
            pltpu.sync_copy(idx_hbm.at[pl.ds(base + c * CHUNK, CHUNK)],
                            idx_v.at[c])
        cps = []
        for c in range(NCH):
            rsel = pl.ds(c * CHUNK, CHUNK)
            for h in range(2):
                cps.append(pltpu.async_copy(
                    table_hbm.at[idx_v.at[c], pl.ds(h * 128, 128)],
                    rows_v.at[rsel, pl.ds(h * 128, 128)], sem))
            cps.append(pltpu.async_copy(
                tail_hbm.at[idx_v.at[c]],
                rows_v.at[rsel, pl.ds(256, 128)], sem))
        for cp in cps:
            cp.wait()
        pltpu.sync_copy(rows_v, out_hbm.at[pl.ds(base, RPW)])

    return k(table, tail_pad, idx_flat)


def _p1(g3, w, b2):
    def body(g_ref, w_ref, b_ref, m_out, d_out, a_out, m_sc, d_sc, a_sc):
        j = pl.program_id(0)

        @pl.when(j == 0)
        def _():
            a_sc[...] = jnp.sum(g_ref[...], axis=0)[:, :EMB]
            m_sc[...] = jnp.full((B, 1), -jnp.inf, jnp.float32)
            d_sc[...] = jnp.zeros((B, 1), jnp.float32)

        logits = lax.dot_general(
            a_sc[...], w_ref[...], (((1,), (0,)), ((), ())),
            preferred_element_type=jnp.float32,
        ) * (1.0 / CTX) + b_ref[...]
        col = j * TV + lax.broadcasted_iota(jnp.int32, (1, TV), 1)
        logits = jnp.where(col < V, logits, -jnp.inf)
        mj = jnp.max(logits, axis=1, keepdims=True)
        nm = jnp.maximum(m_sc[...], mj)
        d_sc[...] = (d_sc[...] * jnp.exp(m_sc[...] - nm)
                     + jnp.sum(jnp.exp(logits - nm), axis=1, keepdims=True))
        m_sc[...] = nm

        @pl.when(j == NV - 1)
        def _():
            m_out[...] = m_sc[...]
            d_out[...] = d_sc[...]
            a_out[...] = a_sc[...]

    return pl.pallas_call(
        body,
        grid=(NV,),
        in_specs=[
            pl.BlockSpec((CTX, B, GEMB), lambda j: (0, 0, 0)),
            pl.BlockSpec((EMB, TV), lambda j: (0, j)),
            pl.BlockSpec((1, TV), lambda j: (0, j)),
        ],
        out_specs=[
            pl.BlockSpec((B, 1), lambda j: (0, 0)),
            pl.BlockSpec((B, 1), lambda j: (0, 0)),
            pl.BlockSpec((B, EMB), lambda j: (0, 0)),
        ],
        out_shape=[
            jax.ShapeDtypeStruct((B, 1), jnp.float32),
            jax.ShapeDtypeStruct((B, 1), jnp.float32),
            jax.ShapeDtypeStruct((B, EMB), jnp.float32),
        ],
        scratch_shapes=[
            pltpu.VMEM((B, 1), jnp.float32),
            pltpu.VMEM((B, 1), jnp.float32),
            pltpu.VMEM((B, EMB), jnp.float32),
        ],
        compiler_params=pltpu.CompilerParams(
            dimension_semantics=("arbitrary",)),
    )(g3, w, b2)


def _p2t(a_sum, w, bcol, m_row, d_row):
    def body(a_ref, w_ref, b_ref, m_ref, d_ref, o_ref):
        lt = lax.dot_general(
            w_ref[...], a_ref[...], (((0,), (1,)), ((), ())),
            preferred_element_type=jnp.float32,
        ) * (1.0 / CTX) + b_ref[...]
        o_ref[...] = jnp.exp(lt - m_ref[...]) * (1.0 / d_ref[...])

    return pl.pallas_call(
        body,
        grid=(NV,),
        in_specs=[
            pl.BlockSpec((B, EMB), lambda j: (0, 0)),
            pl.BlockSpec((EMB, TV), lambda j: (0, j)),
            pl.BlockSpec((TV, 1), lambda j: (j, 0)),
            pl.BlockSpec((1, B), lambda j: (0, 0)),
            pl.BlockSpec((1, B), lambda j: (0, 0)),
        ],
        out_specs=pl.BlockSpec((TV, B), lambda j: (j, 0)),
        out_shape=jax.ShapeDtypeStruct((V, B), jnp.float32),
        compiler_params=pltpu.CompilerParams(
            dimension_semantics=("arbitrary",)),
    )(a_sum, w, bcol, m_row, d_row)


def kernel(inputs, embedding_table, dense_W, dense_b):
    idx_flat = inputs.astype(jnp.int32).T.reshape(-1)
    table_rm = with_layout_constraint(embedding_table, Layout((0, 1)))
    tail_pad = jnp.pad(table_rm[:, 256:], ((0, 0), (0, 128 - (EMB - 256))))
    g = _sc_gather(table_rm, tail_pad, idx_flat)
    g3 = g.reshape(CTX, B, GEMB)
    b2 = dense_b.reshape(1, V)
    m, d, a_sum = _p1(g3, dense_W, b2)
    out_t = _p2t(a_sum, dense_W, dense_b.reshape(V, 1),
                 m.reshape(1, B), d.reshape(1, B))
    return out_t.T
